# Initial kernel scaffold; baseline (speedup 1.0000x reference)
import jax
import jax.numpy as jnp
from jax.experimental import pallas as pl


def kernel(x, edge_index, Wl0, bl0, Wr0, g0, b0, Wl1, bl1, Wr1, g1, b1, Wl2, bl2, Wr2, g2, b2, Ws1, bs1, Ws2, bs2):
    raise NotImplementedError("write your pallas kernel here")


# speedup vs baseline: 4.3946x; 4.3946x over previous
import functools
import math

import jax
import jax.numpy as jnp
from jax import lax
from jax.experimental import pallas as pl
from jax.experimental.pallas import tpu as pltpu
from jax.experimental.pallas import tpu_sc as plsc

_N = 10000
_E = 320000
_D = 128
_EPS = 1e-5
_ISQ = 1.0 / math.sqrt(1.0 + _EPS)

_NC = 2
_NS = 16
_NW = _NC * _NS
_EPW = _E // _NW
_C = 80
_NCH = _EPW // _C
_NP = 10240
_RK = _NP // (_NS * _C)


def _make_sc_kernel(with_gather: bool):
  mesh = plsc.VectorSubcoreMesh(core_axis_name="c", subcore_axis_name="s")
  scratch = [
      pltpu.VMEM((_C,), jnp.int32),
      pltpu.VMEM((_C, _D), jnp.float32),
      pltpu.VMEM_SHARED((_NP, _D), jnp.float32),
      pltpu.SemaphoreType.DMA,
  ]
  if with_gather:
    scratch.insert(0, pltpu.VMEM((_C,), jnp.int32))

  def body(*refs):
    if with_gather:
      (h_hbm, src_hbm, dst_hbm, zcd_hbm, s_out,
       src_v, dst_v, rows_v, acc_sh, sem) = refs
    else:
      (dst_hbm, ones_hbm, zcd_hbm, s_out,
       dst_v, rows_v, acc_sh, sem) = refs
    cid = lax.axis_index("c")
    sid = lax.axis_index("s")
    wid = sid * _NC + cid

    pltpu.sync_copy(zcd_hbm, rows_v)
    for k in range(_RK):
      r0 = (sid * _RK + k) * _C
      pltpu.sync_copy(rows_v, acc_sh.at[pl.ds(r0, _C)])
    plsc.subcore_barrier()

    if not with_gather:
      pltpu.sync_copy(ones_hbm, rows_v)

    def chunk(k, _):
      base = wid * _EPW + k * _C
      pltpu.sync_copy(dst_hbm.at[pl.ds(base, _C)], dst_v)
      if with_gather:
        pltpu.sync_copy(src_hbm.at[pl.ds(base, _C)], src_v)
        pltpu.async_copy(h_hbm.at[src_v], rows_v, sem).wait()
      pltpu.sync_copy(rows_v, acc_sh.at[dst_v], add=True)
      return 0
    lax.fori_loop(0, _NCH, chunk, 0)
    plsc.subcore_barrier()

    for k in range(_RK):
      r0 = (sid * _RK + k) * _C
      pltpu.sync_copy(acc_sh.at[pl.ds(r0, _C)], rows_v)
      pltpu.sync_copy(rows_v, s_out.at[pl.ds(cid * _NP + r0, _C)])

  return pl.kernel(
      body,
      out_type=jax.ShapeDtypeStruct((_NC * _NP, _D), jnp.float32),
      mesh=mesh,
      scratch_types=scratch,
  )


@functools.lru_cache(maxsize=None)
def _get_sc_kernel(with_gather: bool):
  return _make_sc_kernel(with_gather)


def _sc_agg(h, src, dst, zcd):
  return _get_sc_kernel(True)(h, src, dst, zcd)


def _sc_deg(dst, ones_cd, zcd):
  return _get_sc_kernel(False)(dst, ones_cd, zcd)


_BN = 2000


def _make_tc_layer(with_skip: bool):
  def body(*refs):
    if with_skip:
      s2, d2, h, Wl, bl, Wr, g, b, Ws, bs, out = refs
    else:
      s2, d2, h, Wl, bl, Wr, g, b, out = refs
    s = s2[0] + s2[1]
    deg = d2[0][:, :1] + d2[1][:, :1]
    agg = s * (1.0 / jnp.maximum(deg, 1.0))
    pre = (jnp.dot(agg, Wl[...], preferred_element_type=jnp.float32)
           + bl[...]
           + jnp.dot(h[...], Wr[...], preferred_element_type=jnp.float32))
    xn = jnp.maximum(pre * _ISQ * g[...] + b[...], 0.0)
    if with_skip:
      out[...] = (h[...] + jnp.dot(xn, Ws[...], preferred_element_type=jnp.float32)
                  + bs[...])
    else:
      out[...] = xn

  row_block = pl.BlockSpec((_BN, _D), lambda i: (i, 0))
  parts = pl.BlockSpec((_NC, _BN, _D), lambda i: (0, i, 0))
  w_full = pl.BlockSpec((_D, _D), lambda i: (0, 0))
  v_full = pl.BlockSpec((1, _D), lambda i: (0, 0))
  in_specs = [parts, parts, row_block,
              w_full, v_full, w_full, v_full, v_full]
  if with_skip:
    in_specs += [w_full, v_full]
  return pl.pallas_call(
      body,
      grid=(_N // _BN,),
      in_specs=in_specs,
      out_specs=row_block,
      out_shape=jax.ShapeDtypeStruct((_N, _D), jnp.float32),
  )


_tc_layer0 = _make_tc_layer(False)
_tc_layer_skip = _make_tc_layer(True)


def kernel(x, edge_index, Wl0, bl0, Wr0, g0, b0, Wl1, bl1, Wr1, g1, b1,
           Wl2, bl2, Wr2, g2, b2, Ws1, bs1, Ws2, bs2):
  src = edge_index[0]
  dst = edge_index[1]
  r = lambda v: v.reshape(1, _D)
  zcd = jnp.zeros((_C, _D), jnp.float32)
  ones_cd = jnp.ones((_C, _D), jnp.float32)

  deg = _sc_deg(dst, ones_cd, zcd).reshape(_NC, _NP, _D)

  s = _sc_agg(x, src, dst, zcd).reshape(_NC, _NP, _D)
  h1 = _tc_layer0(s, deg, x, Wl0, r(bl0), Wr0, r(g0), r(b0))

  s = _sc_agg(h1, src, dst, zcd).reshape(_NC, _NP, _D)
  h2 = _tc_layer_skip(s, deg, h1, Wl1, r(bl1), Wr1, r(g1), r(b1), Ws1, r(bs1))

  s = _sc_agg(h2, src, dst, zcd).reshape(_NC, _NP, _D)
  h3 = _tc_layer_skip(s, deg, h2, Wl2, r(bl2), Wr2, r(g2), r(b2), Ws2, r(bs2))
  return h3


# speedup vs baseline: 6.6334x; 1.5094x over previous
import functools
import math

import jax
import jax.numpy as jnp
from jax import lax
from jax.experimental import pallas as pl
from jax.experimental.pallas import tpu as pltpu
from jax.experimental.pallas import tpu_sc as plsc

_N = 10000
_E = 320000
_D = 128
_EPS = 1e-5
_ISQ = 1.0 / math.sqrt(1.0 + _EPS)

_NC = 2
_NS = 16
_NW = _NC * _NS
_EPW = _E // _NW
_C = 80
_NCH = _EPW // _C
_NP = 10240
_RK = _NP // (_NS * _C)


def _make_sc_kernel(with_gather: bool):
  mesh = plsc.VectorSubcoreMesh(core_axis_name="c", subcore_axis_name="s")
  scratch = [
      pltpu.VMEM((_C,), jnp.int32),
      pltpu.VMEM((_C, _D), jnp.float32),
      pltpu.VMEM_SHARED((_NP, _D), jnp.float32),
      pltpu.SemaphoreType.DMA,
  ]
  if with_gather:
    scratch = [
        pltpu.VMEM((_C,), jnp.int32),
        pltpu.VMEM((_C,), jnp.int32),
        pltpu.VMEM((_C,), jnp.int32),
        pltpu.VMEM((_C,), jnp.int32),
        pltpu.VMEM((_C, _D), jnp.float32),
        pltpu.VMEM((_C, _D), jnp.float32),
        pltpu.VMEM_SHARED((_NP, _D), jnp.float32),
        pltpu.SemaphoreType.DMA,
        pltpu.SemaphoreType.DMA,
    ]

  def body(*refs):
    if with_gather:
      (h_hbm, src_hbm, dst_hbm, zcd_hbm, s_out,
       src_v0, src_v1, dst_v0, dst_v1, rows_v, rows_v1, acc_sh,
       sem0, sem1) = refs
    else:
      (dst_hbm, ones_hbm, zcd_hbm, s_out,
       dst_v0, rows_v, acc_sh, sem0) = refs
    cid = lax.axis_index("c")
    sid = lax.axis_index("s")
    wid = sid * _NC + cid
    ebase = wid * _EPW

    pltpu.sync_copy(zcd_hbm, rows_v)
    for k in range(_RK):
      r0 = (sid * _RK + k) * _C
      pltpu.sync_copy(rows_v, acc_sh.at[pl.ds(r0, _C)])
    plsc.subcore_barrier()

    if with_gather:
      def load_and_fire(c, src_v, dst_v, rv, sem):
        pltpu.sync_copy(src_hbm.at[pl.ds(ebase + c * _C, _C)], src_v)
        pltpu.sync_copy(dst_hbm.at[pl.ds(ebase + c * _C, _C)], dst_v)
        pltpu.async_copy(h_hbm.at[src_v], rv, sem)

      def wait_and_scatter(src_v, dst_v, rv, sem):
        pltpu.make_async_copy(h_hbm.at[src_v], rv, sem).wait()
        pltpu.sync_copy(rv, acc_sh.at[dst_v], add=True)

      load_and_fire(0, src_v0, dst_v0, rows_v, sem0)

      def pair(j, _):
        load_and_fire(2 * j + 1, src_v1, dst_v1, rows_v1, sem1)
        wait_and_scatter(src_v0, dst_v0, rows_v, sem0)
        load_and_fire(2 * j + 2, src_v0, dst_v0, rows_v, sem0)
        wait_and_scatter(src_v1, dst_v1, rows_v1, sem1)
        return 0
      lax.fori_loop(0, (_NCH - 1) // 2, pair, 0)
      wait_and_scatter(src_v0, dst_v0, rows_v, sem0)
    else:
      pltpu.sync_copy(ones_hbm, rows_v)

      def chunk(k, _):
        pltpu.sync_copy(dst_hbm.at[pl.ds(ebase + k * _C, _C)], dst_v0)
        pltpu.sync_copy(rows_v, acc_sh.at[dst_v0], add=True)
        return 0
      lax.fori_loop(0, _NCH, chunk, 0)
    plsc.subcore_barrier()

    for k in range(_RK):
      r0 = (sid * _RK + k) * _C
      pltpu.sync_copy(acc_sh.at[pl.ds(r0, _C)], rows_v)
      pltpu.sync_copy(rows_v, s_out.at[pl.ds(cid * _NP + r0, _C)])

  return pl.kernel(
      body,
      out_type=jax.ShapeDtypeStruct((_NC * _NP, _D), jnp.float32),
      mesh=mesh,
      scratch_types=scratch,
  )


@functools.lru_cache(maxsize=None)
def _get_sc_kernel(with_gather: bool):
  return _make_sc_kernel(with_gather)


def _sc_agg(h, src, dst, zcd):
  return _get_sc_kernel(True)(h, src, dst, zcd)


def _sc_deg(dst, ones_cd, zcd):
  return _get_sc_kernel(False)(dst, ones_cd, zcd)


_BN = 2000


def _make_tc_layer(with_skip: bool):
  def body(*refs):
    if with_skip:
      s2, d2, h, Wl, bl, Wr, g, b, Ws, bs, out = refs
    else:
      s2, d2, h, Wl, bl, Wr, g, b, out = refs
    s = s2[0] + s2[1]
    deg = d2[0][:, :1] + d2[1][:, :1]
    agg = s * (1.0 / jnp.maximum(deg, 1.0))
    pre = (jnp.dot(agg, Wl[...], preferred_element_type=jnp.float32)
           + bl[...]
           + jnp.dot(h[...], Wr[...], preferred_element_type=jnp.float32))
    xn = jnp.maximum(pre * _ISQ * g[...] + b[...], 0.0)
    if with_skip:
      out[...] = (h[...] + jnp.dot(xn, Ws[...], preferred_element_type=jnp.float32)
                  + bs[...])
    else:
      out[...] = xn

  row_block = pl.BlockSpec((_BN, _D), lambda i: (i, 0))
  parts = pl.BlockSpec((_NC, _BN, _D), lambda i: (0, i, 0))
  w_full = pl.BlockSpec((_D, _D), lambda i: (0, 0))
  v_full = pl.BlockSpec((1, _D), lambda i: (0, 0))
  in_specs = [parts, parts, row_block,
              w_full, v_full, w_full, v_full, v_full]
  if with_skip:
    in_specs += [w_full, v_full]
  return pl.pallas_call(
      body,
      grid=(_N // _BN,),
      in_specs=in_specs,
      out_specs=row_block,
      out_shape=jax.ShapeDtypeStruct((_N, _D), jnp.float32),
  )


_tc_layer0 = _make_tc_layer(False)
_tc_layer_skip = _make_tc_layer(True)


def kernel(x, edge_index, Wl0, bl0, Wr0, g0, b0, Wl1, bl1, Wr1, g1, b1,
           Wl2, bl2, Wr2, g2, b2, Ws1, bs1, Ws2, bs2):
  src = edge_index[0]
  dst = edge_index[1]
  r = lambda v: v.reshape(1, _D)
  zcd = jnp.zeros((_C, _D), jnp.float32)
  ones_cd = jnp.ones((_C, _D), jnp.float32)

  deg = _sc_deg(dst, ones_cd, zcd).reshape(_NC, _NP, _D)

  s = _sc_agg(x, src, dst, zcd).reshape(_NC, _NP, _D)
  h1 = _tc_layer0(s, deg, x, Wl0, r(bl0), Wr0, r(g0), r(b0))

  s = _sc_agg(h1, src, dst, zcd).reshape(_NC, _NP, _D)
  h2 = _tc_layer_skip(s, deg, h1, Wl1, r(bl1), Wr1, r(g1), r(b1), Ws1, r(bs1))

  s = _sc_agg(h2, src, dst, zcd).reshape(_NC, _NP, _D)
  h3 = _tc_layer_skip(s, deg, h2, Wl2, r(bl2), Wr2, r(g2), r(b2), Ws2, r(bs2))
  return h3


# speedup vs baseline: 9.9129x; 1.4944x over previous
import functools
import math

import jax
import jax.numpy as jnp
from jax import lax
from jax.experimental import pallas as pl
from jax.experimental.pallas import tpu as pltpu
from jax.experimental.pallas import tpu_sc as plsc

_N = 10000
_E = 320000
_D = 128
_EPS = 1e-5
_ISQ = 1.0 / math.sqrt(1.0 + _EPS)

_NC = 2
_NS = 16
_NW = _NC * _NS
_EPW = _E // _NW
_C = 80
_NCH = _EPW // _C
_NP = 10240
_RK = _NP // (_NS * _C)


def _make_sc_kernel(with_gather: bool):
  mesh = plsc.VectorSubcoreMesh(core_axis_name="c", subcore_axis_name="s")
  if with_gather:
    scratch = [
        [pltpu.VMEM((_C,), jnp.int32) for _ in range(4)],
        [pltpu.VMEM((_C,), jnp.int32) for _ in range(4)],
        [pltpu.VMEM((_C, _D), jnp.float32) for _ in range(2)],
        pltpu.VMEM_SHARED((_NP, _D), jnp.float32),
        [pltpu.SemaphoreType.DMA for _ in range(4)],
        [pltpu.SemaphoreType.DMA for _ in range(2)],
    ]
  else:
    scratch = [
        [pltpu.VMEM((_C,), jnp.int32) for _ in range(2)],
        pltpu.VMEM((_C, _D), jnp.float32),
        pltpu.VMEM_SHARED((_NP, _D), jnp.float32),
        [pltpu.SemaphoreType.DMA for _ in range(2)],
    ]

  def body(*refs):
    if with_gather:
      (h_hbm, src_hbm, dst_hbm, zcd_hbm, s_out,
       src_v, dst_v, rows, acc_sh, isem, gsem) = refs
      rows_v = rows[0]
    else:
      (dst_hbm, ones_hbm, zcd_hbm, s_out,
       dst_v, rows_v, acc_sh, isem) = refs
    cid = lax.axis_index("c")
    sid = lax.axis_index("s")
    wid = sid * _NC + cid
    ebase = wid * _EPW

    pltpu.sync_copy(zcd_hbm, rows_v)
    for k in range(_RK):
      r0 = (sid * _RK + k) * _C
      pltpu.sync_copy(rows_v, acc_sh.at[pl.ds(r0, _C)])
    plsc.subcore_barrier()

    if with_gather:
      def idx_fire(c, b):
        pltpu.async_copy(src_hbm.at[pl.ds(ebase + c * _C, _C)], src_v[b], isem[b])
        pltpu.async_copy(dst_hbm.at[pl.ds(ebase + c * _C, _C)], dst_v[b], isem[b])

      def idx_wait(c, b):
        pltpu.make_async_copy(src_hbm.at[pl.ds(ebase + c * _C, _C)], src_v[b], isem[b]).wait()
        pltpu.make_async_copy(dst_hbm.at[pl.ds(ebase + c * _C, _C)], dst_v[b], isem[b]).wait()

      def step(c, t):
        b2, b1, b0 = (t + 2) % 4, (t + 1) % 4, t % 4
        r1, r0 = (t + 1) % 2, t % 2
        idx_fire(c + 2, b2)
        idx_wait(c + 1, b1)
        pltpu.async_copy(h_hbm.at[src_v[b1]], rows[r1], gsem[r1])
        pltpu.make_async_copy(h_hbm.at[src_v[b0]], rows[r0], gsem[r0]).wait()
        pltpu.sync_copy(rows[r0], acc_sh.at[dst_v[b0]], add=True)

      idx_fire(0, 0)
      idx_fire(1, 1)
      idx_wait(0, 0)
      pltpu.async_copy(h_hbm.at[src_v[0]], rows[0], gsem[0])

      def quad(j, _):
        for t in range(4):
          step(4 * j + t, t)
        return 0
      nq = (_NCH - 5) // 4
      lax.fori_loop(0, nq, quad, 0)
      for c in range(4 * nq, _NCH):
        t = c % 4
        b1, b0 = (t + 1) % 4, t % 4
        r1, r0 = (t + 1) % 2, t % 2
        if c + 2 < _NCH:
          idx_fire(c + 2, (t + 2) % 4)
        if c + 1 < _NCH:
          idx_wait(c + 1, b1)
          pltpu.async_copy(h_hbm.at[src_v[b1]], rows[r1], gsem[r1])
        pltpu.make_async_copy(h_hbm.at[src_v[b0]], rows[r0], gsem[r0]).wait()
        pltpu.sync_copy(rows[r0], acc_sh.at[dst_v[b0]], add=True)
    else:
      pltpu.sync_copy(ones_hbm, rows_v)

      def idx_fire(c, b):
        pltpu.async_copy(dst_hbm.at[pl.ds(ebase + c * _C, _C)], dst_v[b], isem[b])

      def idx_wait(c, b):
        pltpu.make_async_copy(dst_hbm.at[pl.ds(ebase + c * _C, _C)], dst_v[b], isem[b]).wait()

      idx_fire(0, 0)

      def pair(j, _):
        for t in range(2):
          c = 2 * j + t
          idx_fire(c + 1, (t + 1) % 2)
          idx_wait(c, t)
          pltpu.sync_copy(rows_v, acc_sh.at[dst_v[t]], add=True)
        return 0
      lax.fori_loop(0, (_NCH - 1) // 2, pair, 0)
      idx_wait(_NCH - 1, (_NCH - 1) % 2)
      pltpu.sync_copy(rows_v, acc_sh.at[dst_v[(_NCH - 1) % 2]], add=True)
    plsc.subcore_barrier()

    for k in range(_RK):
      r0 = (sid * _RK + k) * _C
      pltpu.sync_copy(acc_sh.at[pl.ds(r0, _C)], rows_v)
      pltpu.sync_copy(rows_v, s_out.at[pl.ds(cid * _NP + r0, _C)])

  return pl.kernel(
      body,
      out_type=jax.ShapeDtypeStruct((_NC * _NP, _D), jnp.float32),
      mesh=mesh,
      scratch_types=scratch,
  )


@functools.lru_cache(maxsize=None)
def _get_sc_kernel(with_gather: bool):
  return _make_sc_kernel(with_gather)


def _sc_agg(h, src, dst, zcd):
  return _get_sc_kernel(True)(h, src, dst, zcd)


def _sc_deg(dst, ones_cd, zcd):
  return _get_sc_kernel(False)(dst, ones_cd, zcd)


_BN = 2000


def _make_tc_layer(with_skip: bool):
  def body(*refs):
    if with_skip:
      s2, d2, h, Wl, bl, Wr, g, b, Ws, bs, out = refs
    else:
      s2, d2, h, Wl, bl, Wr, g, b, out = refs
    s = s2[0] + s2[1]
    deg = d2[0][:, :1] + d2[1][:, :1]
    agg = s * (1.0 / jnp.maximum(deg, 1.0))
    pre = (jnp.dot(agg, Wl[...], preferred_element_type=jnp.float32)
           + bl[...]
           + jnp.dot(h[...], Wr[...], preferred_element_type=jnp.float32))
    xn = jnp.maximum(pre * _ISQ * g[...] + b[...], 0.0)
    if with_skip:
      out[...] = (h[...] + jnp.dot(xn, Ws[...], preferred_element_type=jnp.float32)
                  + bs[...])
    else:
      out[...] = xn

  row_block = pl.BlockSpec((_BN, _D), lambda i: (i, 0))
  parts = pl.BlockSpec((_NC, _BN, _D), lambda i: (0, i, 0))
  w_full = pl.BlockSpec((_D, _D), lambda i: (0, 0))
  v_full = pl.BlockSpec((1, _D), lambda i: (0, 0))
  in_specs = [parts, parts, row_block,
              w_full, v_full, w_full, v_full, v_full]
  if with_skip:
    in_specs += [w_full, v_full]
  return pl.pallas_call(
      body,
      grid=(_N // _BN,),
      in_specs=in_specs,
      out_specs=row_block,
      out_shape=jax.ShapeDtypeStruct((_N, _D), jnp.float32),
  )


_tc_layer0 = _make_tc_layer(False)
_tc_layer_skip = _make_tc_layer(True)


def kernel(x, edge_index, Wl0, bl0, Wr0, g0, b0, Wl1, bl1, Wr1, g1, b1,
           Wl2, bl2, Wr2, g2, b2, Ws1, bs1, Ws2, bs2):
  src = edge_index[0]
  dst = edge_index[1]
  r = lambda v: v.reshape(1, _D)
  zcd = jnp.zeros((_C, _D), jnp.float32)
  ones_cd = jnp.ones((_C, _D), jnp.float32)

  deg = _sc_deg(dst, ones_cd, zcd).reshape(_NC, _NP, _D)

  s = _sc_agg(x, src, dst, zcd).reshape(_NC, _NP, _D)
  h1 = _tc_layer0(s, deg, x, Wl0, r(bl0), Wr0, r(g0), r(b0))

  s = _sc_agg(h1, src, dst, zcd).reshape(_NC, _NP, _D)
  h2 = _tc_layer_skip(s, deg, h1, Wl1, r(bl1), Wr1, r(g1), r(b1), Ws1, r(bs1))

  s = _sc_agg(h2, src, dst, zcd).reshape(_NC, _NP, _D)
  h3 = _tc_layer_skip(s, deg, h2, Wl2, r(bl2), Wr2, r(g2), r(b2), Ws2, r(bs2))
  return h3


# speedup vs baseline: 10.0034x; 1.0091x over previous
import functools
import math

import jax
import jax.numpy as jnp
from jax import lax
from jax.experimental import pallas as pl
from jax.experimental.pallas import tpu as pltpu
from jax.experimental.pallas import tpu_sc as plsc

_N = 10000
_E = 320000
_D = 128
_EPS = 1e-5
_ISQ = 1.0 / math.sqrt(1.0 + _EPS)

_NC = 2
_NS = 16
_NW = _NC * _NS
_EPW = _E // _NW
_C = 80
_NCH = _EPW // _C
_NP = 10240
_RK = _NP // (_NS * _C)


def _make_sc_kernel(with_gather: bool):
  mesh = plsc.VectorSubcoreMesh(core_axis_name="c", subcore_axis_name="s")
  if with_gather:
    scratch = [
        [pltpu.VMEM((_C,), jnp.int32) for _ in range(4)],
        [pltpu.VMEM((_C,), jnp.int32) for _ in range(4)],
        [pltpu.VMEM((_C, _D), jnp.float32) for _ in range(2)],
        pltpu.VMEM_SHARED((_NP, _D), jnp.float32),
        [pltpu.SemaphoreType.DMA for _ in range(4)],
        [pltpu.SemaphoreType.DMA for _ in range(2)],
    ]
  else:
    scratch = [
        [pltpu.VMEM((_C,), jnp.int32) for _ in range(2)],
        pltpu.VMEM((_C, _D), jnp.float32),
        pltpu.VMEM_SHARED((_NP, _D), jnp.float32),
        [pltpu.SemaphoreType.DMA for _ in range(2)],
    ]

  def body(*refs):
    if with_gather:
      (h_hbm, src_hbm, dst_hbm, zcd_hbm, s_out,
       src_v, dst_v, rows, acc_sh, isem, gsem) = refs
      rows_v = rows[0]
    else:
      (dst_hbm, ones_hbm, zcd_hbm, s_out,
       dst_v, rows_v, acc_sh, isem) = refs
    cid = lax.axis_index("c")
    sid = lax.axis_index("s")
    wid = sid * _NC + cid
    ebase = wid * _EPW

    pltpu.sync_copy(zcd_hbm, rows_v)
    for k in range(_RK):
      r0 = (sid * _RK + k) * _C
      pltpu.sync_copy(rows_v, acc_sh.at[pl.ds(r0, _C)])
    plsc.subcore_barrier()

    if with_gather:
      def idx_fire(c, b):
        pltpu.async_copy(src_hbm.at[pl.ds(ebase + c * _C, _C)], src_v[b], isem[b])
        pltpu.async_copy(dst_hbm.at[pl.ds(ebase + c * _C, _C)], dst_v[b], isem[b])

      def idx_wait(c, b):
        pltpu.make_async_copy(src_hbm.at[pl.ds(ebase + c * _C, _C)], src_v[b], isem[b]).wait()
        pltpu.make_async_copy(dst_hbm.at[pl.ds(ebase + c * _C, _C)], dst_v[b], isem[b]).wait()

      def step(c, t):
        b2, b1, b0 = (t + 2) % 4, (t + 1) % 4, t % 4
        r1, r0 = (t + 1) % 2, t % 2
        idx_fire(c + 2, b2)
        idx_wait(c + 1, b1)
        pltpu.async_copy(h_hbm.at[src_v[b1]], rows[r1], gsem[r1])
        pltpu.make_async_copy(h_hbm.at[src_v[b0]], rows[r0], gsem[r0]).wait()
        pltpu.sync_copy(rows[r0], acc_sh.at[dst_v[b0]], add=True)

      idx_fire(0, 0)
      idx_fire(1, 1)
      idx_wait(0, 0)
      pltpu.async_copy(h_hbm.at[src_v[0]], rows[0], gsem[0])

      def quad(j, _):
        for t in range(4):
          step(4 * j + t, t)
        return 0
      nq = (_NCH - 5) // 4
      lax.fori_loop(0, nq, quad, 0)
      for c in range(4 * nq, _NCH):
        t = c % 4
        b1, b0 = (t + 1) % 4, t % 4
        r1, r0 = (t + 1) % 2, t % 2
        if c + 2 < _NCH:
          idx_fire(c + 2, (t + 2) % 4)
        if c + 1 < _NCH:
          idx_wait(c + 1, b1)
          pltpu.async_copy(h_hbm.at[src_v[b1]], rows[r1], gsem[r1])
        pltpu.make_async_copy(h_hbm.at[src_v[b0]], rows[r0], gsem[r0]).wait()
        pltpu.sync_copy(rows[r0], acc_sh.at[dst_v[b0]], add=True)
    else:
      pltpu.sync_copy(ones_hbm, rows_v)

      def idx_fire(c, b):
        pltpu.async_copy(dst_hbm.at[pl.ds(ebase + c * _C, _C)], dst_v[b], isem[b])

      def idx_wait(c, b):
        pltpu.make_async_copy(dst_hbm.at[pl.ds(ebase + c * _C, _C)], dst_v[b], isem[b]).wait()

      idx_fire(0, 0)

      def pair(j, _):
        for t in range(2):
          c = 2 * j + t
          idx_fire(c + 1, (t + 1) % 2)
          idx_wait(c, t)
          pltpu.sync_copy(rows_v, acc_sh.at[dst_v[t]], add=True)
        return 0
      lax.fori_loop(0, (_NCH - 1) // 2, pair, 0)
      idx_wait(_NCH - 1, (_NCH - 1) % 2)
      pltpu.sync_copy(rows_v, acc_sh.at[dst_v[(_NCH - 1) % 2]], add=True)
    plsc.subcore_barrier()

    for k in range(_RK):
      r0 = (sid * _RK + k) * _C
      pltpu.sync_copy(acc_sh.at[pl.ds(r0, _C)], s_out.at[pl.ds(cid * _NP + r0, _C)])

  return pl.kernel(
      body,
      out_type=jax.ShapeDtypeStruct((_NC * _NP, _D), jnp.float32),
      mesh=mesh,
      scratch_types=scratch,
  )


@functools.lru_cache(maxsize=None)
def _get_sc_kernel(with_gather: bool):
  return _make_sc_kernel(with_gather)


def _sc_agg(h, src, dst, zcd):
  return _get_sc_kernel(True)(h, src, dst, zcd)


def _sc_deg(dst, ones_cd, zcd):
  return _get_sc_kernel(False)(dst, ones_cd, zcd)


_BN = 2000


def _make_tc_layer(with_skip: bool):
  def body(*refs):
    if with_skip:
      s2, d2, h, Wl, bl, Wr, g, b, Ws, bs, out = refs
    else:
      s2, d2, h, Wl, bl, Wr, g, b, out = refs
    s = s2[0] + s2[1]
    deg = d2[0][:, :1] + d2[1][:, :1]
    agg = s * (1.0 / jnp.maximum(deg, 1.0))
    pre = (jnp.dot(agg, Wl[...], preferred_element_type=jnp.float32)
           + bl[...]
           + jnp.dot(h[...], Wr[...], preferred_element_type=jnp.float32))
    xn = jnp.maximum(pre * _ISQ * g[...] + b[...], 0.0)
    if with_skip:
      out[...] = (h[...] + jnp.dot(xn, Ws[...], preferred_element_type=jnp.float32)
                  + bs[...])
    else:
      out[...] = xn

  row_block = pl.BlockSpec((_BN, _D), lambda i: (i, 0))
  parts = pl.BlockSpec((_NC, _BN, _D), lambda i: (0, i, 0))
  w_full = pl.BlockSpec((_D, _D), lambda i: (0, 0))
  v_full = pl.BlockSpec((1, _D), lambda i: (0, 0))
  in_specs = [parts, parts, row_block,
              w_full, v_full, w_full, v_full, v_full]
  if with_skip:
    in_specs += [w_full, v_full]
  return pl.pallas_call(
      body,
      grid=(_N // _BN,),
      in_specs=in_specs,
      out_specs=row_block,
      out_shape=jax.ShapeDtypeStruct((_N, _D), jnp.float32),
  )


_tc_layer0 = _make_tc_layer(False)
_tc_layer_skip = _make_tc_layer(True)


def kernel(x, edge_index, Wl0, bl0, Wr0, g0, b0, Wl1, bl1, Wr1, g1, b1,
           Wl2, bl2, Wr2, g2, b2, Ws1, bs1, Ws2, bs2):
  src = edge_index[0]
  dst = edge_index[1]
  r = lambda v: v.reshape(1, _D)
  zcd = jnp.zeros((_C, _D), jnp.float32)
  ones_cd = jnp.ones((_C, _D), jnp.float32)

  deg = _sc_deg(dst, ones_cd, zcd).reshape(_NC, _NP, _D)

  s = _sc_agg(x, src, dst, zcd).reshape(_NC, _NP, _D)
  h1 = _tc_layer0(s, deg, x, Wl0, r(bl0), Wr0, r(g0), r(b0))

  s = _sc_agg(h1, src, dst, zcd).reshape(_NC, _NP, _D)
  h2 = _tc_layer_skip(s, deg, h1, Wl1, r(bl1), Wr1, r(g1), r(b1), Ws1, r(bs1))

  s = _sc_agg(h2, src, dst, zcd).reshape(_NC, _NP, _D)
  h3 = _tc_layer_skip(s, deg, h2, Wl2, r(bl2), Wr2, r(g2), r(b2), Ws2, r(bs2))
  return h3


# speedup vs baseline: 10.0070x; 1.0004x over previous
import functools
import math

import jax
import jax.numpy as jnp
from jax import lax
from jax.experimental import pallas as pl
from jax.experimental.pallas import tpu as pltpu
from jax.experimental.pallas import tpu_sc as plsc

_N = 10000
_E = 320000
_D = 128
_EPS = 1e-5
_ISQ = 1.0 / math.sqrt(1.0 + _EPS)

_NC = 2
_NS = 16
_NW = _NC * _NS
_EPW = _E // _NW
_C = 80
_NCH = _EPW // _C
_NP = 10240
_RK = _NP // (_NS * _C)


def _make_sc_kernel(with_gather: bool):
  mesh = plsc.VectorSubcoreMesh(core_axis_name="c", subcore_axis_name="s")
  if with_gather:
    scratch = [
        [pltpu.VMEM((_C,), jnp.int32) for _ in range(4)],
        [pltpu.VMEM((_C,), jnp.int32) for _ in range(4)],
        [pltpu.VMEM((_C, _D), jnp.float32) for _ in range(2)],
        pltpu.VMEM_SHARED((_NP, _D), jnp.float32),
        [pltpu.SemaphoreType.DMA for _ in range(4)],
        [pltpu.SemaphoreType.DMA for _ in range(2)],
    ]
  else:
    scratch = [
        [pltpu.VMEM((_C,), jnp.int32) for _ in range(2)],
        pltpu.VMEM((_C, _D), jnp.float32),
        pltpu.VMEM_SHARED((_NP, _D), jnp.float32),
        [pltpu.SemaphoreType.DMA for _ in range(2)],
    ]

  def body(*refs):
    if with_gather:
      (h_hbm, src_hbm, dst_hbm, zcd_hbm, s_out,
       src_v, dst_v, rows, acc_sh, isem, gsem) = refs
      rows_v = rows[0]
    else:
      (dst_hbm, ones_hbm, zcd_hbm, s_out,
       dst_v, rows_v, acc_sh, isem) = refs
    cid = lax.axis_index("c")
    sid = lax.axis_index("s")
    wid = sid * _NC + cid
    ebase = wid * _EPW

    pltpu.sync_copy(zcd_hbm, rows_v)
    for k in range(_RK):
      r0 = (sid * _RK + k) * _C
      pltpu.sync_copy(rows_v, acc_sh.at[pl.ds(r0, _C)])
    plsc.subcore_barrier()

    if with_gather:
      def idx_fire(c, b):
        pltpu.async_copy(src_hbm.at[pl.ds(ebase + c * _C, _C)], src_v[b], isem[b])
        pltpu.async_copy(dst_hbm.at[pl.ds(ebase + c * _C, _C)], dst_v[b], isem[b])

      def idx_wait(c, b):
        pltpu.make_async_copy(src_hbm.at[pl.ds(ebase + c * _C, _C)], src_v[b], isem[b]).wait()
        pltpu.make_async_copy(dst_hbm.at[pl.ds(ebase + c * _C, _C)], dst_v[b], isem[b]).wait()

      def step(c, t):
        b2, b1, b0 = (t + 2) % 4, (t + 1) % 4, t % 4
        r1, r0 = (t + 1) % 2, t % 2
        idx_fire(c + 2, b2)
        idx_wait(c + 1, b1)
        pltpu.async_copy(h_hbm.at[src_v[b1]], rows[r1], gsem[r1])
        pltpu.make_async_copy(h_hbm.at[src_v[b0]], rows[r0], gsem[r0]).wait()
        pltpu.sync_copy(rows[r0], acc_sh.at[dst_v[b0]], add=True)

      idx_fire(0, 0)
      idx_fire(1, 1)
      idx_wait(0, 0)
      pltpu.async_copy(h_hbm.at[src_v[0]], rows[0], gsem[0])

      def quad(j, _):
        for t in range(4):
          step(4 * j + t, t)
        return 0
      nq = (_NCH - 5) // 4
      lax.fori_loop(0, nq, quad, 0)
      for c in range(4 * nq, _NCH):
        t = c % 4
        b1, b0 = (t + 1) % 4, t % 4
        r1, r0 = (t + 1) % 2, t % 2
        if c + 2 < _NCH:
          idx_fire(c + 2, (t + 2) % 4)
        if c + 1 < _NCH:
          idx_wait(c + 1, b1)
          pltpu.async_copy(h_hbm.at[src_v[b1]], rows[r1], gsem[r1])
        pltpu.make_async_copy(h_hbm.at[src_v[b0]], rows[r0], gsem[r0]).wait()
        pltpu.sync_copy(rows[r0], acc_sh.at[dst_v[b0]], add=True)
    else:
      pltpu.sync_copy(ones_hbm, rows_v)

      def idx_fire(c, b):
        pltpu.async_copy(dst_hbm.at[pl.ds(ebase + c * _C, _C)], dst_v[b], isem[b])

      def idx_wait(c, b):
        pltpu.make_async_copy(dst_hbm.at[pl.ds(ebase + c * _C, _C)], dst_v[b], isem[b]).wait()

      idx_fire(0, 0)

      def pair(j, _):
        for t in range(2):
          c = 2 * j + t
          idx_fire(c + 1, (t + 1) % 2)
          idx_wait(c, t)
          pltpu.sync_copy(rows_v, acc_sh.at[dst_v[t]], add=True)
        return 0
      lax.fori_loop(0, (_NCH - 1) // 2, pair, 0)
      idx_wait(_NCH - 1, (_NCH - 1) % 2)
      pltpu.sync_copy(rows_v, acc_sh.at[dst_v[(_NCH - 1) % 2]], add=True)
    plsc.subcore_barrier()

    for k in range(_RK):
      r0 = (sid * _RK + k) * _C
      pltpu.sync_copy(acc_sh.at[pl.ds(r0, _C)], s_out.at[pl.ds(cid * _NP + r0, _C)])

  return pl.kernel(
      body,
      out_type=jax.ShapeDtypeStruct((_NC * _NP, _D), jnp.float32),
      mesh=mesh,
      scratch_types=scratch,
  )


@functools.lru_cache(maxsize=None)
def _get_sc_kernel(with_gather: bool):
  return _make_sc_kernel(with_gather)


def _sc_agg(h, src, dst, zcd):
  return _get_sc_kernel(True)(h, src, dst, zcd)


def _sc_deg(dst, ones_cd, zcd):
  return _get_sc_kernel(False)(dst, ones_cd, zcd)


_BN = 2000


def _make_tc_layer(with_skip: bool):
  def body(*refs):
    if with_skip:
      s2, d2, h, Wl, bl, Wr, g, b, Ws, bs, out = refs
    else:
      s2, d2, h, Wl, bl, Wr, g, b, out = refs
    s = s2[0] + s2[1]
    deg = d2[0][:, :1] + d2[1][:, :1]
    agg = s * (1.0 / jnp.maximum(deg, 1.0))
    pre = (jnp.dot(agg, Wl[...], preferred_element_type=jnp.float32)
           + bl[...]
           + jnp.dot(h[...], Wr[...], preferred_element_type=jnp.float32))
    xn = jnp.maximum(pre * _ISQ * g[...] + b[...], 0.0)
    if with_skip:
      out[...] = (h[...] + jnp.dot(xn, Ws[...], preferred_element_type=jnp.float32)
                  + bs[...])
    else:
      out[...] = xn

  row_block = pl.BlockSpec((_BN, _D), lambda i: (i, 0))
  parts = pl.BlockSpec((_NC, _BN, _D), lambda i: (0, i, 0))
  dparts = pl.BlockSpec((_NC, _BN, 16), lambda i: (0, i, 0))
  w_full = pl.BlockSpec((_D, _D), lambda i: (0, 0))
  v_full = pl.BlockSpec((1, _D), lambda i: (0, 0))
  in_specs = [parts, dparts, row_block,
              w_full, v_full, w_full, v_full, v_full]
  if with_skip:
    in_specs += [w_full, v_full]
  return pl.pallas_call(
      body,
      grid=(_N // _BN,),
      in_specs=in_specs,
      out_specs=row_block,
      out_shape=jax.ShapeDtypeStruct((_N, _D), jnp.float32),
  )


_tc_layer0 = _make_tc_layer(False)
_tc_layer_skip = _make_tc_layer(True)


def kernel(x, edge_index, Wl0, bl0, Wr0, g0, b0, Wl1, bl1, Wr1, g1, b1,
           Wl2, bl2, Wr2, g2, b2, Ws1, bs1, Ws2, bs2):
  src = edge_index[0]
  dst = edge_index[1]
  r = lambda v: v.reshape(1, _D)
  zcd = jnp.zeros((_C, _D), jnp.float32)
  ones_cd = jnp.ones((_C, _D), jnp.float32)

  deg = _sc_deg(dst, ones_cd, zcd).reshape(_NC, _NP, _D)[:, :, :16]

  s = _sc_agg(x, src, dst, zcd).reshape(_NC, _NP, _D)
  h1 = _tc_layer0(s, deg, x, Wl0, r(bl0), Wr0, r(g0), r(b0))

  s = _sc_agg(h1, src, dst, zcd).reshape(_NC, _NP, _D)
  h2 = _tc_layer_skip(s, deg, h1, Wl1, r(bl1), Wr1, r(g1), r(b1), Ws1, r(bs1))

  s = _sc_agg(h2, src, dst, zcd).reshape(_NC, _NP, _D)
  h3 = _tc_layer_skip(s, deg, h2, Wl2, r(bl2), Wr2, r(g2), r(b2), Ws2, r(bs2))
  return h3


# speedup vs baseline: 10.0192x; 1.0012x over previous
import functools
import math

import jax
import jax.numpy as jnp
from jax import lax
from jax.experimental import pallas as pl
from jax.experimental.pallas import tpu as pltpu
from jax.experimental.pallas import tpu_sc as plsc

_N = 10000
_E = 320000
_D = 128
_EPS = 1e-5
_ISQ = 1.0 / math.sqrt(1.0 + _EPS)

_NC = 2
_NS = 16
_NW = _NC * _NS
_EPW = _E // _NW
_CG = 40
_NCHG = _EPW // _CG
_CD = 80
_NCHD = _EPW // _CD
_NP = 10240


def _make_sc_kernel(with_gather: bool):
  mesh = plsc.VectorSubcoreMesh(core_axis_name="c", subcore_axis_name="s")
  C = _CG if with_gather else _CD
  NCH = _NCHG if with_gather else _NCHD
  RK = _NP // (_NS * C)
  if with_gather:
    scratch = [
        [pltpu.VMEM((C,), jnp.int32) for _ in range(8)],
        [pltpu.VMEM((C,), jnp.int32) for _ in range(8)],
        [pltpu.VMEM((C, _D), jnp.float32) for _ in range(4)],
        pltpu.VMEM_SHARED((_NP, _D), jnp.float32),
        [pltpu.SemaphoreType.DMA for _ in range(8)],
        [pltpu.SemaphoreType.DMA for _ in range(4)],
    ]
  else:
    scratch = [
        [pltpu.VMEM((C,), jnp.int32) for _ in range(2)],
        pltpu.VMEM((C, _D), jnp.float32),
        pltpu.VMEM_SHARED((_NP, _D), jnp.float32),
        [pltpu.SemaphoreType.DMA for _ in range(2)],
    ]

  def body(*refs):
    if with_gather:
      (h_hbm, src_hbm, dst_hbm, zcd_hbm, s_out,
       src_v, dst_v, rows, acc_sh, isem, gsem) = refs
      rows_v = rows[0]
    else:
      (dst_hbm, ones_hbm, zcd_hbm, s_out,
       dst_v, rows_v, acc_sh, isem) = refs
    cid = lax.axis_index("c")
    sid = lax.axis_index("s")
    wid = sid * _NC + cid
    ebase = wid * _EPW

    pltpu.sync_copy(zcd_hbm, rows_v)
    for k in range(RK):
      r0 = (sid * RK + k) * C
      pltpu.sync_copy(rows_v, acc_sh.at[pl.ds(r0, C)])
    plsc.subcore_barrier()

    if with_gather:
      def idx_fire(c, b):
        pltpu.async_copy(src_hbm.at[pl.ds(ebase + c * C, C)], src_v[b], isem[b])
        pltpu.async_copy(dst_hbm.at[pl.ds(ebase + c * C, C)], dst_v[b], isem[b])

      def idx_wait(c, b):
        pltpu.make_async_copy(src_hbm.at[pl.ds(ebase + c * C, C)], src_v[b], isem[b]).wait()
        pltpu.make_async_copy(dst_hbm.at[pl.ds(ebase + c * C, C)], dst_v[b], isem[b]).wait()

      def step(c, t, last=NCH):
        if not isinstance(c, int) or c + 5 < last:
          idx_fire(c + 5, (t + 5) % 8)
        if not isinstance(c, int) or c + 4 < last:
          idx_wait(c + 4, (t + 4) % 8)
        if not isinstance(c, int) or c + 3 < last:
          b3 = (t + 3) % 8
          pltpu.async_copy(h_hbm.at[src_v[b3]], rows[(t + 3) % 4], gsem[(t + 3) % 4])
        pltpu.make_async_copy(h_hbm.at[src_v[t % 8]], rows[t % 4], gsem[t % 4]).wait()
        pltpu.sync_copy(rows[t % 4], acc_sh.at[dst_v[t % 8]], add=True)

      for k in range(5):
        idx_fire(k, k)
      for k in range(4):
        idx_wait(k, k)
      for k in range(3):
        pltpu.async_copy(h_hbm.at[src_v[k]], rows[k], gsem[k])

      def oct8(j, _):
        for t in range(8):
          step(8 * j + t, t)
        return 0
      nq = (NCH - 10) // 8
      lax.fori_loop(0, nq, oct8, 0)
      for c in range(8 * nq, NCH):
        step(c, c % 8)
    else:
      pltpu.sync_copy(ones_hbm, rows_v)

      def didx_fire(c, b):
        pltpu.async_copy(dst_hbm.at[pl.ds(ebase + c * C, C)], dst_v[b], isem[b])

      def didx_wait(c, b):
        pltpu.make_async_copy(dst_hbm.at[pl.ds(ebase + c * C, C)], dst_v[b], isem[b]).wait()

      didx_fire(0, 0)

      def pair(j, _):
        for t in range(2):
          c = 2 * j + t
          didx_fire(c + 1, (t + 1) % 2)
          didx_wait(c, t)
          pltpu.sync_copy(rows_v, acc_sh.at[dst_v[t]], add=True)
        return 0
      lax.fori_loop(0, (NCH - 1) // 2, pair, 0)
      didx_wait(NCH - 1, (NCH - 1) % 2)
      pltpu.sync_copy(rows_v, acc_sh.at[dst_v[(NCH - 1) % 2]], add=True)
    plsc.subcore_barrier()

    for k in range(RK):
      r0 = (sid * RK + k) * C
      pltpu.sync_copy(acc_sh.at[pl.ds(r0, C)], s_out.at[pl.ds(cid * _NP + r0, C)])

  return pl.kernel(
      body,
      out_type=jax.ShapeDtypeStruct((_NC * _NP, _D), jnp.float32),
      mesh=mesh,
      scratch_types=scratch,
  )


@functools.lru_cache(maxsize=None)
def _get_sc_kernel(with_gather: bool):
  return _make_sc_kernel(with_gather)


def _sc_agg(h, src, dst, zcd):
  return _get_sc_kernel(True)(h, src, dst, zcd)


def _sc_deg(dst, ones_cd, zcd):
  return _get_sc_kernel(False)(dst, ones_cd, zcd)


_BN = 2000


def _make_tc_layer(with_skip: bool):
  def body(*refs):
    if with_skip:
      s2, d2, h, Wl, bl, Wr, g, b, Ws, bs, out = refs
    else:
      s2, d2, h, Wl, bl, Wr, g, b, out = refs
    s = s2[0] + s2[1]
    deg = d2[0][:, :1] + d2[1][:, :1]
    agg = s * (1.0 / jnp.maximum(deg, 1.0))
    pre = (jnp.dot(agg, Wl[...], preferred_element_type=jnp.float32)
           + bl[...]
           + jnp.dot(h[...], Wr[...], preferred_element_type=jnp.float32))
    xn = jnp.maximum(pre * _ISQ * g[...] + b[...], 0.0)
    if with_skip:
      out[...] = (h[...] + jnp.dot(xn, Ws[...], preferred_element_type=jnp.float32)
                  + bs[...])
    else:
      out[...] = xn

  row_block = pl.BlockSpec((_BN, _D), lambda i: (i, 0))
  parts = pl.BlockSpec((_NC, _BN, _D), lambda i: (0, i, 0))
  dparts = pl.BlockSpec((_NC, _BN, 16), lambda i: (0, i, 0))
  w_full = pl.BlockSpec((_D, _D), lambda i: (0, 0))
  v_full = pl.BlockSpec((1, _D), lambda i: (0, 0))
  in_specs = [parts, dparts, row_block,
              w_full, v_full, w_full, v_full, v_full]
  if with_skip:
    in_specs += [w_full, v_full]
  return pl.pallas_call(
      body,
      grid=(_N // _BN,),
      in_specs=in_specs,
      out_specs=row_block,
      out_shape=jax.ShapeDtypeStruct((_N, _D), jnp.float32),
  )


_tc_layer0 = _make_tc_layer(False)
_tc_layer_skip = _make_tc_layer(True)


def kernel(x, edge_index, Wl0, bl0, Wr0, g0, b0, Wl1, bl1, Wr1, g1, b1,
           Wl2, bl2, Wr2, g2, b2, Ws1, bs1, Ws2, bs2):
  src = edge_index[0]
  dst = edge_index[1]
  r = lambda v: v.reshape(1, _D)
  zcd_g = jnp.zeros((_CG, _D), jnp.float32)
  zcd_d = jnp.zeros((_CD, _D), jnp.float32)
  ones_cd = jnp.ones((_CD, _D), jnp.float32)

  deg = _sc_deg(dst, ones_cd, zcd_d).reshape(_NC, _NP, _D)[:, :, :16]

  s = _sc_agg(x, src, dst, zcd_g).reshape(_NC, _NP, _D)
  h1 = _tc_layer0(s, deg, x, Wl0, r(bl0), Wr0, r(g0), r(b0))

  s = _sc_agg(h1, src, dst, zcd_g).reshape(_NC, _NP, _D)
  h2 = _tc_layer_skip(s, deg, h1, Wl1, r(bl1), Wr1, r(g1), r(b1), Ws1, r(bs1))

  s = _sc_agg(h2, src, dst, zcd_g).reshape(_NC, _NP, _D)
  h3 = _tc_layer_skip(s, deg, h2, Wl2, r(bl2), Wr2, r(g2), r(b2), Ws2, r(bs2))
  return h3
